# baseline (speedup 1.0000x reference)
import functools

import jax
import jax.numpy as jnp
from jax import lax
from jax.experimental import pallas as pl
from jax.experimental.pallas import tpu as pltpu
from jax.experimental.pallas import tpu_sc as plsc

DIM = 64
N = 4096 * 200
NC = 2
NS = 16
NW = NC * NS
PER_W = N // NW
C = 800
NCH = PER_W // C

_mesh = plsc.VectorSubcoreMesh(core_axis_name="c", subcore_axis_name="s")


@functools.partial(
    pl.kernel,
    mesh=_mesh,
    out_type=jax.ShapeDtypeStruct((N, DIM), jnp.float32),
    scratch_types=[
        pltpu.VMEM((C,), jnp.int32),
        pltpu.VMEM((C, DIM), jnp.float32),
        pltpu.SemaphoreType.DMA,
    ],
    compiler_params=pltpu.CompilerParams(use_tc_tiling_on_sc=False),
)
def _emb_lookup(idx_hbm, table_hbm, out_hbm, idx_v, rows_v, sem):
    wid = lax.axis_index("s") * NC + lax.axis_index("c")
    base = wid * PER_W

    def body(g, carry):
        off = pl.multiple_of(base + g * C, 8)
        pltpu.sync_copy(idx_hbm.at[pl.ds(off, C)], idx_v)
        pltpu.async_copy(table_hbm.at[idx_v], rows_v, sem).wait()
        pltpu.sync_copy(rows_v, out_hbm.at[pl.ds(off, C)])
        return carry

    lax.fori_loop(0, NCH, body, 0)


def kernel(token_ids, table):
    flat = token_ids.reshape(-1).astype(jnp.int32)
    out = _emb_lookup(flat, table)
    return out.reshape(token_ids.shape + (DIM,))


# speedup vs baseline: 1.0242x; 1.0242x over previous
import functools

import jax
import jax.numpy as jnp
from jax import lax
from jax.experimental import pallas as pl
from jax.experimental.pallas import tpu as pltpu
from jax.experimental.pallas import tpu_sc as plsc

DIM = 64
N = 4096 * 200
NC = 2
NS = 16
NW = NC * NS
PER_W = N // NW
C = 640
NCH = PER_W // C
NCH2 = NCH // 2

_mesh = plsc.VectorSubcoreMesh(core_axis_name="c", subcore_axis_name="s")


@functools.partial(
    pl.kernel,
    mesh=_mesh,
    out_type=jax.ShapeDtypeStruct((N, DIM), jnp.float32),
    scratch_types=[
        pltpu.VMEM((PER_W,), jnp.int32),
        pltpu.VMEM((C, DIM), jnp.float32),
        pltpu.VMEM((C, DIM), jnp.float32),
        pltpu.SemaphoreType.DMA,
        pltpu.SemaphoreType.DMA,
        pltpu.SemaphoreType.DMA,
    ],
    compiler_params=pltpu.CompilerParams(use_tc_tiling_on_sc=False),
)
def _emb_lookup(idx_hbm, table_hbm, out_hbm, idx_all, rows_v0, rows_v1,
                gsem, ssem0, ssem1):
    wid = lax.axis_index("s") * NC + lax.axis_index("c")
    base = wid * PER_W
    pltpu.sync_copy(idx_hbm.at[pl.ds(pl.multiple_of(base, 8), PER_W)], idx_all)

    def body(i, carry):
        e = 2 * i
        eo = pl.multiple_of(e * C, 8)
        oo = pl.multiple_of((e + 1) * C, 8)
        ebase = pl.multiple_of(base + e * C, 8)
        obase = pl.multiple_of(base + (e + 1) * C, 8)

        @pl.when(i > 0)
        def _():
            pltpu.make_async_copy(rows_v0, out_hbm.at[pl.ds(0, C)], ssem0).wait()

        pltpu.async_copy(
            table_hbm.at[idx_all.at[pl.ds(eo, C)]], rows_v0, gsem).wait()
        pltpu.async_copy(rows_v0, out_hbm.at[pl.ds(ebase, C)], ssem0)

        @pl.when(i > 0)
        def _():
            pltpu.make_async_copy(rows_v1, out_hbm.at[pl.ds(0, C)], ssem1).wait()

        pltpu.async_copy(
            table_hbm.at[idx_all.at[pl.ds(oo, C)]], rows_v1, gsem).wait()
        pltpu.async_copy(rows_v1, out_hbm.at[pl.ds(obase, C)], ssem1)
        return carry

    lax.fori_loop(0, NCH2, body, 0)
    pltpu.make_async_copy(rows_v0, out_hbm.at[pl.ds(0, C)], ssem0).wait()
    pltpu.make_async_copy(rows_v1, out_hbm.at[pl.ds(0, C)], ssem1).wait()


def kernel(token_ids, table):
    flat = token_ids.reshape(-1).astype(jnp.int32)
    out = _emb_lookup(flat, table)
    return out.reshape(token_ids.shape + (DIM,))
